# baseline (speedup 1.0000x reference)
import functools

import jax
import jax.numpy as jnp
from jax import lax
from jax.experimental import pallas as pl
from jax.experimental.pallas import tpu as pltpu
from jax.experimental.pallas import tpu_sc as plsc

NUM_CORES = 2
NUM_SUBCORES = 16
NUM_WORKERS = NUM_CORES * NUM_SUBCORES
BATCH = 16384
BPW = BATCH // NUM_WORKERS
G = 16
NGROUPS = BPW // G
DEPTH = 3
D = 5
LANES = 16
W = 128
STRIP = G * W


def _issue_group(pt_hbm, qt_hbm, uid_v, iid_v, pbuf, qbuf, sems_p, sems_q, g):
    d = g % DEPTH
    uvec = uid_v[pl.ds(g * G, G)]
    ivec = iid_v[pl.ds(g * G, G)]
    ubase = (uvec // W) * W
    ibase = (ivec // W) * W
    for k in range(G):
        su = pl.multiple_of(ubase[k], W)
        si = pl.multiple_of(ibase[k], W)
        dst = pl.ds(d * STRIP + k * W, W)
        pltpu.async_copy(pt_hbm.at[:, pl.ds(su, W)], pbuf.at[:, dst], sems_p[d])
        pltpu.async_copy(qt_hbm.at[:, pl.ds(si, W)], qbuf.at[:, dst], sems_q[d])


def _drain_group(pt_hbm, qt_hbm, pbuf, qbuf, sems_p, sems_q, g):
    d = g % DEPTH
    strip = pl.ds(d * STRIP, STRIP)
    pltpu.make_async_copy(pt_hbm.at[:, pl.ds(0, STRIP)], pbuf.at[:, strip],
                          sems_p[d]).wait()
    pltpu.make_async_copy(qt_hbm.at[:, pl.ds(0, STRIP)], qbuf.at[:, strip],
                          sems_q[d]).wait()


def _process_group(uid_v, iid_v, pbuf, qbuf, g, acc):
    d = g % DEPTH
    iota = lax.iota(jnp.int32, LANES)
    uvec = uid_v[pl.ds(g * G, G)]
    ivec = iid_v[pl.ds(g * G, G)]
    uoff = d * STRIP + iota * W + lax.rem(uvec, jnp.int32(W))
    ioff = d * STRIP + iota * W + lax.rem(ivec, jnp.int32(W))
    for c in range(D):
        cs = jnp.full((LANES,), c, jnp.int32)
        pv = plsc.load_gather(pbuf, [cs, uoff])
        qv = plsc.load_gather(qbuf, [cs, ioff])
        acc = acc + pv * qv
    return acc


def _partials_body(pt_hbm, qt_hbm, uid_hbm, iid_hbm, out_hbm,
                   uid_v, iid_v, pbuf, qbuf, acc_v,
                   sem_p0, sem_p1, sem_p2, sem_q0, sem_q1, sem_q2):
    sems_p = (sem_p0, sem_p1, sem_p2)
    sems_q = (sem_q0, sem_q1, sem_q2)
    wid = lax.axis_index("s") * NUM_CORES + lax.axis_index("c")
    pltpu.sync_copy(uid_hbm.at[pl.ds(wid * BPW, BPW)], uid_v)
    pltpu.sync_copy(iid_hbm.at[pl.ds(wid * BPW, BPW)], iid_v)

    for g in range(DEPTH):
        _issue_group(pt_hbm, qt_hbm, uid_v, iid_v, pbuf, qbuf, sems_p, sems_q, g)

    acc = jnp.zeros((LANES,), jnp.float32)
    for g in range(NGROUPS):
        _drain_group(pt_hbm, qt_hbm, pbuf, qbuf, sems_p, sems_q, g)
        acc = _process_group(uid_v, iid_v, pbuf, qbuf, g, acc)
        if g + DEPTH < NGROUPS:
            _issue_group(pt_hbm, qt_hbm, uid_v, iid_v, pbuf, qbuf,
                         sems_p, sems_q, g + DEPTH)

    acc_v[...] = acc
    pltpu.sync_copy(acc_v, out_hbm.at[wid])


_lfm_partials = functools.partial(
    pl.kernel,
    out_type=jax.ShapeDtypeStruct((NUM_WORKERS, LANES), jnp.float32),
    mesh=plsc.VectorSubcoreMesh(core_axis_name="c", subcore_axis_name="s",
                                num_cores=NUM_CORES, num_subcores=NUM_SUBCORES),
    compiler_params=pltpu.CompilerParams(needs_layout_passes=False,
                                         use_tc_tiling_on_sc=True),
    scratch_types=[
        pltpu.VMEM((BPW,), jnp.int32),
        pltpu.VMEM((BPW,), jnp.int32),
        pltpu.VMEM((8, DEPTH * STRIP), jnp.float32),
        pltpu.VMEM((D, DEPTH * STRIP), jnp.float32),
        pltpu.VMEM((LANES,), jnp.float32),
        pltpu.SemaphoreType.DMA,
        pltpu.SemaphoreType.DMA,
        pltpu.SemaphoreType.DMA,
        pltpu.SemaphoreType.DMA,
        pltpu.SemaphoreType.DMA,
        pltpu.SemaphoreType.DMA,
    ],
)(_partials_body)


def _finish_body(x_ref, o_ref):
    r = jnp.sum(x_ref[...])
    o_ref[0, 0] = 1.0 / (1.0 + jnp.exp(-r))


def kernel(P, Q, user_id, item_id):
    PT8 = jnp.pad(P.T, ((0, 3), (0, 0)))
    partials = _lfm_partials(PT8, Q.T,
                             user_id.astype(jnp.int32),
                             item_id.astype(jnp.int32))
    out = pl.pallas_call(
        _finish_body,
        out_shape=jax.ShapeDtypeStruct((1, 1), jnp.float32),
        out_specs=pl.BlockSpec(memory_space=pltpu.SMEM),
    )(partials)
    return out[0, 0]


# speedup vs baseline: 1.0035x; 1.0035x over previous
import functools

import jax
import jax.numpy as jnp
from jax import lax
from jax.experimental import pallas as pl
from jax.experimental.pallas import tpu as pltpu
from jax.experimental.pallas import tpu_sc as plsc

NUM_CORES = 2
NUM_SUBCORES = 16
NUM_WORKERS = NUM_CORES * NUM_SUBCORES
BATCH = 16384
BPW = BATCH // NUM_WORKERS
G = 16
NGROUPS = BPW // G
DEPTH = 3
D = 5
LANES = 16
W = 128
STRIP = G * W


def _issue_group(pt_hbm, qt_hbm, uid_v, iid_v, pbuf, qbuf, sems_p, sems_q, g):
    d = g % DEPTH
    uvec = uid_v[pl.ds(g * G, G)]
    ivec = iid_v[pl.ds(g * G, G)]
    ubase = (uvec // W) * W
    ibase = (ivec // W) * W
    for k in range(G):
        su = pl.multiple_of(ubase[k], W)
        si = pl.multiple_of(ibase[k], W)
        dst = pl.ds(d * STRIP + k * W, W)
        pltpu.async_copy(pt_hbm.at[:, pl.ds(su, W)], pbuf.at[:, dst], sems_p[d])
        pltpu.async_copy(qt_hbm.at[:, pl.ds(si, W)], qbuf.at[:, dst], sems_q[d])


def _drain_group(pt_hbm, qt_hbm, pbuf, qbuf, sems_p, sems_q, g):
    d = g % DEPTH
    strip = pl.ds(d * STRIP, STRIP)
    pltpu.make_async_copy(pt_hbm.at[:, pl.ds(0, STRIP)], pbuf.at[:, strip],
                          sems_p[d]).wait()
    pltpu.make_async_copy(qt_hbm.at[:, pl.ds(0, STRIP)], qbuf.at[:, strip],
                          sems_q[d]).wait()


def _process_group(uid_v, iid_v, pbuf, qbuf, g, acc):
    d = g % DEPTH
    iota = lax.iota(jnp.int32, LANES)
    uvec = uid_v[pl.ds(g * G, G)]
    ivec = iid_v[pl.ds(g * G, G)]
    uoff = d * STRIP + iota * W + lax.rem(uvec, jnp.int32(W))
    ioff = d * STRIP + iota * W + lax.rem(ivec, jnp.int32(W))
    for c in range(D):
        cs = jnp.full((LANES,), c, jnp.int32)
        pv = plsc.load_gather(pbuf, [cs, uoff])
        qv = plsc.load_gather(qbuf, [cs, ioff])
        acc = acc + pv * qv
    return acc


def _partials_body(pt_hbm, qt_hbm, uid_hbm, iid_hbm, out_hbm,
                   uid_v, iid_v, pbuf, qbuf, acc_v,
                   sem_p0, sem_p1, sem_p2, sem_q0, sem_q1, sem_q2):
    sems_p = (sem_p0, sem_p1, sem_p2)
    sems_q = (sem_q0, sem_q1, sem_q2)
    wid = lax.axis_index("s") * NUM_CORES + lax.axis_index("c")
    pltpu.sync_copy(uid_hbm.at[pl.ds(wid * BPW, BPW)], uid_v)
    pltpu.sync_copy(iid_hbm.at[pl.ds(wid * BPW, BPW)], iid_v)

    for g in range(DEPTH):
        _issue_group(pt_hbm, qt_hbm, uid_v, iid_v, pbuf, qbuf, sems_p, sems_q, g)

    acc = jnp.zeros((LANES,), jnp.float32)
    for g in range(NGROUPS):
        _drain_group(pt_hbm, qt_hbm, pbuf, qbuf, sems_p, sems_q, g)
        acc = _process_group(uid_v, iid_v, pbuf, qbuf, g, acc)
        if g + DEPTH < NGROUPS:
            _issue_group(pt_hbm, qt_hbm, uid_v, iid_v, pbuf, qbuf,
                         sems_p, sems_q, g + DEPTH)

    acc_v[...] = acc
    pltpu.sync_copy(acc_v, out_hbm.at[wid])


_lfm_partials = functools.partial(
    pl.kernel,
    out_type=jax.ShapeDtypeStruct((NUM_WORKERS, LANES), jnp.float32),
    mesh=plsc.VectorSubcoreMesh(core_axis_name="c", subcore_axis_name="s",
                                num_cores=NUM_CORES, num_subcores=NUM_SUBCORES),
    compiler_params=pltpu.CompilerParams(needs_layout_passes=False,
                                         use_tc_tiling_on_sc=True),
    scratch_types=[
        pltpu.VMEM((BPW,), jnp.int32),
        pltpu.VMEM((BPW,), jnp.int32),
        pltpu.VMEM((8, DEPTH * STRIP), jnp.float32),
        pltpu.VMEM((D, DEPTH * STRIP), jnp.float32),
        pltpu.VMEM((LANES,), jnp.float32),
        pltpu.SemaphoreType.DMA,
        pltpu.SemaphoreType.DMA,
        pltpu.SemaphoreType.DMA,
        pltpu.SemaphoreType.DMA,
        pltpu.SemaphoreType.DMA,
        pltpu.SemaphoreType.DMA,
    ],
)(_partials_body)


def _finish_body(x_hbm, o_ref, x_vmem, sem):
    pltpu.async_copy(x_hbm, x_vmem, sem).wait()
    r = jnp.sum(x_vmem[...])
    o_ref[0, 0] = 1.0 / (1.0 + jnp.exp(-r))


def kernel(P, Q, user_id, item_id):
    PT8 = jnp.pad(P.T, ((0, 3), (0, 0)))
    partials = _lfm_partials(PT8, Q.T,
                             user_id.astype(jnp.int32),
                             item_id.astype(jnp.int32))
    out = pl.pallas_call(
        _finish_body,
        out_shape=jax.ShapeDtypeStruct((1, 1), jnp.float32),
        in_specs=[pl.BlockSpec(memory_space=pl.ANY)],
        out_specs=pl.BlockSpec(memory_space=pltpu.SMEM),
        scratch_shapes=[
            pltpu.VMEM((NUM_WORKERS, LANES), jnp.float32),
            pltpu.SemaphoreType.DMA,
        ],
    )(partials)
    return out[0, 0]
